# baseline (device time: 198337 ns/iter reference)
import jax
import jax.numpy as jnp
from jax import lax
from jax.experimental import pallas as pl
from jax.experimental.pallas import tpu as pltpu

N_DEV = 32
N_CHUNK = 8


def kernel(x, pi):
    _, m, n = x.shape
    mh = m // 2
    mr = m - mh
    rows = mr // N_CHUNK

    def body(x_ref, pi_ref, out_ref, relay_ref,
             s_dir, r_dir, s_rel, r_rel, s_fwd, r_fwd):
        my_i = lax.axis_index("i")
        dst = pi_ref[my_i]
        relay = lax.rem(my_i + 1, N_DEV)
        fwd_dst = lax.rem(dst + N_DEV - 1, N_DEV)

        rdma_rel = []
        for c in range(N_CHUNK):
            r = pltpu.make_async_remote_copy(
                src_ref=x_ref.at[:, pl.ds(mh + c * rows, rows), :],
                dst_ref=relay_ref.at[:, pl.ds(c * rows, rows), :],
                send_sem=s_rel.at[c],
                recv_sem=r_rel.at[c],
                device_id=(relay,),
                device_id_type=pl.DeviceIdType.MESH,
            )
            r.start()
            rdma_rel.append(r)

        rdma_dir = pltpu.make_async_remote_copy(
            src_ref=x_ref.at[:, pl.ds(0, mh), :],
            dst_ref=out_ref.at[:, pl.ds(0, mh), :],
            send_sem=s_dir,
            recv_sem=r_dir,
            device_id=(dst,),
            device_id_type=pl.DeviceIdType.MESH,
        )
        rdma_dir.start()

        rdma_fwd = []
        for c in range(N_CHUNK):
            rdma_rel[c].wait_recv()
            f = pltpu.make_async_remote_copy(
                src_ref=relay_ref.at[:, pl.ds(c * rows, rows), :],
                dst_ref=out_ref.at[:, pl.ds(mh + c * rows, rows), :],
                send_sem=s_fwd.at[c],
                recv_sem=r_fwd.at[c],
                device_id=(fwd_dst,),
                device_id_type=pl.DeviceIdType.MESH,
            )
            f.start()
            rdma_fwd.append(f)

        rdma_dir.wait()
        for c in range(N_CHUNK):
            rdma_rel[c].wait_send()
            rdma_fwd[c].wait()

    out_shape = jax.ShapeDtypeStruct(x.shape, jnp.float32)
    return pl.pallas_call(
        body,
        out_shape=out_shape,
        in_specs=[
            pl.BlockSpec(memory_space=pltpu.VMEM),
            pl.BlockSpec(memory_space=pltpu.SMEM),
        ],
        out_specs=pl.BlockSpec(memory_space=pltpu.VMEM),
        scratch_shapes=[
            pltpu.VMEM((1, mr, n), jnp.float32),
            pltpu.SemaphoreType.DMA,
            pltpu.SemaphoreType.DMA,
            pltpu.SemaphoreType.DMA((N_CHUNK,)),
            pltpu.SemaphoreType.DMA((N_CHUNK,)),
            pltpu.SemaphoreType.DMA((N_CHUNK,)),
            pltpu.SemaphoreType.DMA((N_CHUNK,)),
        ],
    )(x, pi)


# device time: 156645 ns/iter; 1.2662x vs baseline; 1.2662x over previous
import jax
import jax.numpy as jnp
from jax import lax
from jax.experimental import pallas as pl
from jax.experimental.pallas import tpu as pltpu

N_DEV = 32
N_CHUNK = 8


def kernel(x, pi):
    _, m, n = x.shape
    mh = m // 2
    mr = m - mh
    rows = mr // N_CHUNK

    def body(x_ref, pi_ref, out_ref, relay_ref,
             s_dir, r_dir, s_rel, r_rel, s_fwd, r_fwd):
        my_i = lax.axis_index("i")
        dst = pi_ref[my_i]
        relay = lax.rem(my_i + 1, N_DEV)
        fwd_dst = lax.rem(dst + N_DEV - 1, N_DEV)

        rdma_dir = pltpu.make_async_remote_copy(
            src_ref=x_ref.at[:, pl.ds(0, mh), :],
            dst_ref=out_ref.at[:, pl.ds(0, mh), :],
            send_sem=s_dir,
            recv_sem=r_dir,
            device_id=(dst,),
            device_id_type=pl.DeviceIdType.MESH,
        )
        rdma_dir.start()

        rdma_rel = []
        for c in range(N_CHUNK):
            r = pltpu.make_async_remote_copy(
                src_ref=x_ref.at[:, pl.ds(mh + c * rows, rows), :],
                dst_ref=relay_ref.at[:, pl.ds(c * rows, rows), :],
                send_sem=s_rel.at[c],
                recv_sem=r_rel.at[c],
                device_id=(relay,),
                device_id_type=pl.DeviceIdType.MESH,
            )
            r.start()
            rdma_rel.append(r)

        rdma_fwd = []
        for c in range(N_CHUNK):
            rdma_rel[c].wait_recv()
            f = pltpu.make_async_remote_copy(
                src_ref=relay_ref.at[:, pl.ds(c * rows, rows), :],
                dst_ref=out_ref.at[:, pl.ds(mh + c * rows, rows), :],
                send_sem=s_fwd.at[c],
                recv_sem=r_fwd.at[c],
                device_id=(fwd_dst,),
                device_id_type=pl.DeviceIdType.MESH,
            )
            f.start()
            rdma_fwd.append(f)

        rdma_dir.wait()
        for c in range(N_CHUNK):
            rdma_rel[c].wait_send()
            rdma_fwd[c].wait()

    out_shape = jax.ShapeDtypeStruct(x.shape, jnp.float32)
    return pl.pallas_call(
        body,
        out_shape=out_shape,
        in_specs=[
            pl.BlockSpec(memory_space=pltpu.VMEM),
            pl.BlockSpec(memory_space=pltpu.SMEM),
        ],
        out_specs=pl.BlockSpec(memory_space=pltpu.VMEM),
        scratch_shapes=[
            pltpu.VMEM((1, mr, n), jnp.float32),
            pltpu.SemaphoreType.DMA,
            pltpu.SemaphoreType.DMA,
            pltpu.SemaphoreType.DMA((N_CHUNK,)),
            pltpu.SemaphoreType.DMA((N_CHUNK,)),
            pltpu.SemaphoreType.DMA((N_CHUNK,)),
            pltpu.SemaphoreType.DMA((N_CHUNK,)),
        ],
    )(x, pi)


# device time: 155220 ns/iter; 1.2778x vs baseline; 1.0092x over previous
import functools

import jax
import jax.numpy as jnp
from jax import lax
from jax.experimental import pallas as pl
from jax.experimental.pallas import tpu as pltpu

N_DEV = 32
N_CQ = 2


def kernel(x, pi):
    _, m, n = x.shape
    mh = m // 2
    mq = mh // 2
    rows = mq // N_CQ

    def body(x_ref, pi_ref, out_ref, relay_ref, s_dir, r_dir,
             s_rel1, r_rel1, s_rel2, r_rel2,
             s_fwd1, r_fwd1, s_fwd2, r_fwd2):
        my_i = lax.axis_index("i")
        dst = pi_ref[my_i]
        right = lax.rem(my_i + 1, N_DEV)
        dstm1 = lax.rem(dst + N_DEV - 1, N_DEV)

        s = lax.rem(my_i - dst + N_DEV, N_DEV)
        src_dir = lax.rem(my_i + s, N_DEV)
        left = lax.rem(my_i + N_DEV - 1, N_DEV)
        src_b2 = lax.rem(my_i + s + 1, N_DEV)
        partners = [dst, right, dstm1, src_dir, left, src_b2]
        barrier = pltpu.get_barrier_semaphore()
        for p in partners:
            pl.semaphore_signal(
                barrier, inc=1,
                device_id=(p,), device_id_type=pl.DeviceIdType.MESH,
            )
        pl.semaphore_wait(barrier, len(partners))

        rdma_dir = pltpu.make_async_remote_copy(
            src_ref=x_ref.at[:, pl.ds(0, mh), :],
            dst_ref=out_ref.at[:, pl.ds(0, mh), :],
            send_sem=s_dir,
            recv_sem=r_dir,
            device_id=(dst,),
            device_id_type=pl.DeviceIdType.MESH,
        )
        rdma_dir.start()

        rdma_rel1 = []
        for c in range(N_CQ):
            r = pltpu.make_async_remote_copy(
                src_ref=x_ref.at[:, pl.ds(mh + c * rows, rows), :],
                dst_ref=relay_ref.at[:, pl.ds(c * rows, rows), :],
                send_sem=s_rel1.at[c],
                recv_sem=r_rel1.at[c],
                device_id=(right,),
                device_id_type=pl.DeviceIdType.MESH,
            )
            r.start()
            rdma_rel1.append(r)

        rdma_rel2 = []
        for c in range(N_CQ):
            r = pltpu.make_async_remote_copy(
                src_ref=x_ref.at[:, pl.ds(mh + mq + c * rows, rows), :],
                dst_ref=relay_ref.at[:, pl.ds(mq + c * rows, rows), :],
                send_sem=s_rel2.at[c],
                recv_sem=r_rel2.at[c],
                device_id=(dstm1,),
                device_id_type=pl.DeviceIdType.MESH,
            )
            r.start()
            rdma_rel2.append(r)

        rdma_fwd1 = []
        for c in range(N_CQ):
            rdma_rel1[c].wait_recv()
            f = pltpu.make_async_remote_copy(
                src_ref=relay_ref.at[:, pl.ds(c * rows, rows), :],
                dst_ref=out_ref.at[:, pl.ds(mh + c * rows, rows), :],
                send_sem=s_fwd1.at[c],
                recv_sem=r_fwd1.at[c],
                device_id=(dstm1,),
                device_id_type=pl.DeviceIdType.MESH,
            )
            f.start()
            rdma_fwd1.append(f)

        rdma_fwd2 = []
        for c in range(N_CQ):
            rdma_rel2[c].wait_recv()
            f = pltpu.make_async_remote_copy(
                src_ref=relay_ref.at[:, pl.ds(mq + c * rows, rows), :],
                dst_ref=out_ref.at[:, pl.ds(mh + mq + c * rows, rows), :],
                send_sem=s_fwd2.at[c],
                recv_sem=r_fwd2.at[c],
                device_id=(right,),
                device_id_type=pl.DeviceIdType.MESH,
            )
            f.start()
            rdma_fwd2.append(f)

        rdma_dir.wait()
        for c in range(N_CQ):
            rdma_rel1[c].wait_send()
            rdma_rel2[c].wait_send()
            rdma_fwd1[c].wait()
            rdma_fwd2[c].wait()

        @functools.partial(
            pl.run_scoped, exit_sem=pltpu.SemaphoreType.REGULAR
        )
        def _(exit_sem):
            for p in partners:
                pl.semaphore_signal(
                    exit_sem, inc=1,
                    device_id=(p,), device_id_type=pl.DeviceIdType.MESH,
                )
            pl.semaphore_wait(exit_sem, len(partners))

    out_shape = jax.ShapeDtypeStruct(x.shape, jnp.float32)
    return pl.pallas_call(
        body,
        out_shape=out_shape,
        in_specs=[
            pl.BlockSpec(memory_space=pl.ANY),
            pl.BlockSpec(memory_space=pltpu.SMEM),
        ],
        out_specs=pl.BlockSpec(memory_space=pl.ANY),
        scratch_shapes=[
            pltpu.VMEM((1, 2 * mq, n), jnp.float32),
            pltpu.SemaphoreType.DMA,
            pltpu.SemaphoreType.DMA,
            pltpu.SemaphoreType.DMA((N_CQ,)),
            pltpu.SemaphoreType.DMA((N_CQ,)),
            pltpu.SemaphoreType.DMA((N_CQ,)),
            pltpu.SemaphoreType.DMA((N_CQ,)),
            pltpu.SemaphoreType.DMA((N_CQ,)),
            pltpu.SemaphoreType.DMA((N_CQ,)),
            pltpu.SemaphoreType.DMA((N_CQ,)),
            pltpu.SemaphoreType.DMA((N_CQ,)),
        ],
        compiler_params=pltpu.CompilerParams(collective_id=0),
    )(x, pi)
